# Initial kernel scaffold; baseline (speedup 1.0000x reference)
import jax
import jax.numpy as jnp
from jax.experimental import pallas as pl


def kernel(x, edge_index, edge_attr, W_in, b_in, W_nb0, b_nb0, W_self0, b_self0, W_nb1, b_nb1, W_self1, b_self1, W_jump, b_jump, W_sc1, b_sc1, W_sc2, b_sc2, W_as1, b_as1, W_as2, b_as2):
    raise NotImplementedError("write your pallas kernel here")


# speedup vs baseline: 2.7931x; 2.7931x over previous
import functools

import jax
import jax.numpy as jnp
from jax import lax
from jax.experimental import pallas as pl
from jax.experimental.pallas import tpu as pltpu
import jax.experimental.pallas.tpu_sc as plsc

N = 10000
NF = 128
EF = 16
H = 256
M = 15

NSC = 2
NTILE = 16
K = 128
COLW = H // 2
HALF = N // NSC
ACC_ROWS = ((HALF + 8 + 127) // 128) * 128
TRASH = HALF
INIT_STRIPE = ACC_ROWS // NTILE
OUT_STRIPE = (HALF // NTILE) // 8 * 8
OUT_LAST = HALF - (NTILE - 1) * OUT_STRIPE
ROW_BLK = 1000


def _segsum_body(with_edge, *refs):
    if with_edge:
        (src_hbm, dst_hbm, ea_hbm, ha_hbm, hb_hbm, zh_hbm,
         agga_hbm, aggb_hbm, eacnt_hbm,
         idx_s, idx_d, dln, rows, eav, acc_h, sem) = refs
    else:
        (src_hbm, dst_hbm, ha_hbm, hb_hbm, zh_hbm,
         agga_hbm, aggb_hbm,
         idx_s, idx_d, dln, rows, eav, acc_h, sem) = refs

    cid = lax.axis_index("c")
    sid = lax.axis_index("s")
    lo = cid * HALF
    hi = lo + HALF
    r0 = sid * INIT_STRIPE

    ept = src_hbm.shape[0] // NTILE
    nchunk = ept // K

    passes = [(ha_hbm, agga_hbm, False), (hb_hbm, aggb_hbm, False)]
    if with_edge:
        passes.append((ha_hbm, eacnt_hbm, True))

    for h_hbm, out_hbm, ea_pass in passes:
        pltpu.sync_copy(zh_hbm.at[pl.ds(r0, INIT_STRIPE)],
                        acc_h.at[pl.ds(r0, INIT_STRIPE)])
        if ea_pass:
            one_v = jnp.ones((16,), jnp.float32)
            zero_v = jnp.zeros((16,), jnp.float32)

            def initrow(r, _):
                rows[r, pl.ds(16, 16)] = one_v
                for c in range(2, 8):
                    rows[r, pl.ds(c * 16, 16)] = zero_v
                return 0

            lax.fori_loop(0, K, initrow, 0)
        plsc.subcore_barrier()

        def chunk(i, _):
            base = sid * ept + i * K
            pltpu.sync_copy(dst_hbm.at[pl.ds(base, K)], idx_d)
            for j in range(K // 16):
                d = idx_d[pl.ds(j * 16, 16)]
                in_half = (d >= lo) & (d < hi)
                dln[pl.ds(j * 16, 16)] = jnp.where(in_half, d - lo, TRASH)
            if ea_pass:
                pltpu.sync_copy(ea_hbm.at[pl.ds(base, K)], eav)

                def fill(r, _):
                    rows[r, pl.ds(0, EF)] = eav[r, pl.ds(0, EF)]
                    return 0

                lax.fori_loop(0, K, fill, 0)
            else:
                pltpu.sync_copy(src_hbm.at[pl.ds(base, K)], idx_s)
                pltpu.async_copy(h_hbm.at[idx_s], rows, sem).wait()
            pltpu.sync_copy(rows, acc_h.at[dln], add=True)
            return 0

        lax.fori_loop(0, nchunk, chunk, 0)
        plsc.subcore_barrier()

        @pl.when(sid < NTILE - 1)
        def _():
            pltpu.sync_copy(acc_h.at[pl.ds(sid * OUT_STRIPE, OUT_STRIPE)],
                            out_hbm.at[pl.ds(lo + sid * OUT_STRIPE, OUT_STRIPE)])

        @pl.when(sid == NTILE - 1)
        def _():
            pltpu.sync_copy(
                acc_h.at[pl.ds((NTILE - 1) * OUT_STRIPE, OUT_LAST)],
                out_hbm.at[pl.ds(lo + (NTILE - 1) * OUT_STRIPE, OUT_LAST)])
        plsc.subcore_barrier()


@functools.cache
def _make_segsum(with_edge):
    mesh = plsc.VectorSubcoreMesh(core_axis_name="c", subcore_axis_name="s",
                                  num_cores=NSC, num_subcores=NTILE)
    half = jax.ShapeDtypeStruct((N, COLW), jnp.float32)
    if with_edge:
        out_type = (half, half, half)
    else:
        out_type = (half, half)
    scratch = [
        pltpu.VMEM((K,), jnp.int32),
        pltpu.VMEM((K,), jnp.int32),
        pltpu.VMEM((K,), jnp.int32),
        pltpu.VMEM((K, COLW), jnp.float32),
        pltpu.VMEM((K, EF), jnp.float32),
        pltpu.VMEM_SHARED((ACC_ROWS, COLW), jnp.float32),
        pltpu.SemaphoreType.DMA,
    ]
    return pl.kernel(functools.partial(_segsum_body, with_edge),
                     out_type=out_type, mesh=mesh, scratch_types=scratch,
                     name=f"sage_segsum_{'ea' if with_edge else 'h'}")


def _in_proj_body(x_ref, w_ref, b_ref, o_ref, oa_ref, ob_ref):
    h = jnp.dot(x_ref[...], w_ref[...],
                preferred_element_type=jnp.float32) + b_ref[...]
    o_ref[...] = h
    oa_ref[...] = h[:, :COLW]
    ob_ref[...] = h[:, COLW:]


def _layer_body(h_ref, agga_ref, aggb_ref, eacnt_ref, wnh_ref,
                wne_ref, ws_ref, bb_ref, o_ref, oa_ref, ob_ref):
    eacnt = eacnt_ref[...]
    cnt = eacnt[:, EF:EF + 1] + 1.0
    agg = jnp.concatenate([agga_ref[...], aggb_ref[...]], axis=1)
    pre = (agg + h_ref[...]) / cnt
    q = (jnp.dot(pre, wnh_ref[...], preferred_element_type=jnp.float32)
         + jnp.dot(eacnt[:, :EF] / cnt, wne_ref[...],
                   preferred_element_type=jnp.float32)
         + jnp.dot(h_ref[...], ws_ref[...], preferred_element_type=jnp.float32)
         + bb_ref[...])
    nrm = jnp.sqrt(jnp.sum(q * q, axis=1, keepdims=True))
    out = jnp.maximum(q / jnp.maximum(nrm, 1e-12), 0.0)
    o_ref[...] = out
    oa_ref[...] = out[:, :COLW]
    ob_ref[...] = out[:, COLW:]


def _head_body(hcat_ref, wj_ref, bj_ref, w1_ref, b1_ref, w2_ref, b2_ref,
               wa1_ref, ba1_ref, wa2_ref, ba2_ref, o_ref):
    emb = jnp.dot(hcat_ref[...], wj_ref[...],
                  preferred_element_type=jnp.float32) + bj_ref[...]
    t = jnp.maximum(jnp.dot(emb, w1_ref[...],
                            preferred_element_type=jnp.float32) + b1_ref[...],
                    0.0)
    sc = jnp.sum(t * w2_ref[...].reshape(1, -1), axis=1,
                 keepdims=True) + b2_ref[0, 0]
    ridx = lax.broadcasted_iota(jnp.int32, (16, 1), 0)
    srow = jnp.where(ridx == 0, jnp.float32(-10.0), sc)
    hid = jnp.sum(srow * wa1_ref[...], axis=0, keepdims=True)
    u = jnp.maximum(hid + ba1_ref[...], 0.0)
    o_ref[...] = jnp.dot(u, wa2_ref[...],
                         preferred_element_type=jnp.float32) + ba2_ref[...]


_HALF_SPEC = pl.BlockSpec((ROW_BLK, COLW), lambda i: (i, 0))
_FULL_SPEC = pl.BlockSpec((ROW_BLK, H), lambda i: (i, 0))


def kernel(x, edge_index, edge_attr, W_in, b_in, W_nb0, b_nb0, W_self0, b_self0,
           W_nb1, b_nb1, W_self1, b_self1, W_jump, b_jump,
           W_sc1, b_sc1, W_sc2, b_sc2, W_as1, b_as1, W_as2, b_as2):
    E = edge_index.shape[1]
    ept = ((E + NTILE * K - 1) // (NTILE * K)) * K
    e_pad = ept * NTILE

    src = jnp.concatenate([edge_index[0].astype(jnp.int32),
                           jnp.zeros((e_pad - E,), jnp.int32)])
    dst = jnp.concatenate([edge_index[1].astype(jnp.int32),
                           jnp.full((e_pad - E,), 2**30, jnp.int32)])
    ea = jnp.concatenate([edge_attr,
                          jnp.zeros((e_pad - E, EF), jnp.float32)], axis=0)
    zh = jnp.zeros((ACC_ROWS, COLW), jnp.float32)

    half_sd = jax.ShapeDtypeStruct((N, COLW), jnp.float32)
    full_sd = jax.ShapeDtypeStruct((N, H), jnp.float32)

    h, ha, hb = pl.pallas_call(
        _in_proj_body,
        grid=(N // ROW_BLK,),
        in_specs=[pl.BlockSpec((ROW_BLK, NF), lambda i: (i, 0)),
                  pl.BlockSpec((NF, H), lambda i: (0, 0)),
                  pl.BlockSpec((1, H), lambda i: (0, 0))],
        out_specs=[_FULL_SPEC, _HALF_SPEC, _HALF_SPEC],
        out_shape=[full_sd, half_sd, half_sd],
    )(x, W_in, b_in[None, :])

    def layer(hcur, agga, aggb, eacnt, W_nb, b_nb, W_self, b_self):
        bb = (b_nb + b_self)[None, :]
        return pl.pallas_call(
            _layer_body,
            grid=(N // ROW_BLK,),
            in_specs=[_FULL_SPEC, _HALF_SPEC, _HALF_SPEC, _HALF_SPEC,
                      pl.BlockSpec((H, H), lambda i: (0, 0)),
                      pl.BlockSpec((EF, H), lambda i: (0, 0)),
                      pl.BlockSpec((H, H), lambda i: (0, 0)),
                      pl.BlockSpec((1, H), lambda i: (0, 0))],
            out_specs=[_FULL_SPEC, _HALF_SPEC, _HALF_SPEC],
            out_shape=[full_sd, half_sd, half_sd],
        )(hcur, agga, aggb, eacnt, W_nb[:H], W_nb[H:], W_self, bb)

    agg0a, agg0b, eacnt = _make_segsum(True)(src, dst, ea, ha, hb, zh)
    h1, h1a, h1b = layer(h, agg0a, agg0b, eacnt,
                         W_nb0, b_nb0, W_self0, b_self0)

    agg1a, agg1b = _make_segsum(False)(src, dst, h1a, h1b, zh)
    h2, _, _ = layer(h1, agg1a, agg1b, eacnt,
                     W_nb1, b_nb1, W_self1, b_self1)

    hcat = jnp.concatenate([h1[:16], h2[:16]], axis=1)
    wa1 = jnp.concatenate([W_as1, jnp.zeros((1, H // 2), jnp.float32)], axis=0)
    logits = pl.pallas_call(
        _head_body,
        in_specs=[pl.BlockSpec((16, 2 * H), lambda: (0, 0)),
                  pl.BlockSpec((2 * H, H), lambda: (0, 0)),
                  pl.BlockSpec((1, H), lambda: (0, 0)),
                  pl.BlockSpec((H, H // 2), lambda: (0, 0)),
                  pl.BlockSpec((1, H // 2), lambda: (0, 0)),
                  pl.BlockSpec((H // 2, 1), lambda: (0, 0)),
                  pl.BlockSpec((1, 1), lambda: (0, 0)),
                  pl.BlockSpec((16, H // 2), lambda: (0, 0)),
                  pl.BlockSpec((1, H // 2), lambda: (0, 0)),
                  pl.BlockSpec((H // 2, M), lambda: (0, 0)),
                  pl.BlockSpec((1, M), lambda: (0, 0))],
        out_specs=pl.BlockSpec((1, M), lambda: (0, 0)),
        out_shape=jax.ShapeDtypeStruct((1, M), jnp.float32),
    )(hcat, W_jump, b_jump[None, :], W_sc1, b_sc1[None, :], W_sc2,
      b_sc2[None, :], wa1, b_as1[None, :], W_as2, b_as2[None, :])
    return logits


# speedup vs baseline: 3.6097x; 1.2924x over previous
import functools

import jax
import jax.numpy as jnp
from jax import lax
from jax.experimental import pallas as pl
from jax.experimental.pallas import tpu as pltpu
import jax.experimental.pallas.tpu_sc as plsc

N = 10000
NF = 128
EF = 16
H = 256
M = 15

NSC = 2
NTILE = 16
K = 128
COLW = H // 2
HALF = N // NSC
ACC_ROWS = ((HALF + 8 + 127) // 128) * 128
TRASH = HALF
INIT_STRIPE = ACC_ROWS // NTILE
OUT_STRIPE = (HALF // NTILE) // 8 * 8
OUT_LAST = HALF - (NTILE - 1) * OUT_STRIPE
ROW_BLK = 1000


def _segsum_body(with_edge, *refs):
    if with_edge:
        (src_hbm, dst_hbm, ea_hbm, ha_hbm, hb_hbm, zh_hbm,
         agga_hbm, aggb_hbm, eacnt_hbm,
         idx_s, idx_d, dln, rows, eav, acc_h, sem0, sem1) = refs
    else:
        (src_hbm, dst_hbm, ha_hbm, hb_hbm, zh_hbm,
         agga_hbm, aggb_hbm,
         idx_s, idx_d, dln, rows, eav, acc_h, sem0, sem1) = refs
    gsems = (sem0, sem1)

    cid = lax.axis_index("c")
    sid = lax.axis_index("s")
    lo = cid * HALF
    hi = lo + HALF
    r0 = sid * INIT_STRIPE

    ept = src_hbm.shape[0] // NTILE
    nchunk = ept // K

    passes = [(ha_hbm, agga_hbm, False), (hb_hbm, aggb_hbm, False)]
    if with_edge:
        passes.append((ha_hbm, eacnt_hbm, True))

    for h_hbm, out_hbm, ea_pass in passes:
        pltpu.sync_copy(zh_hbm.at[pl.ds(r0, INIT_STRIPE)],
                        acc_h.at[pl.ds(r0, INIT_STRIPE)])
        if ea_pass:
            one_v = jnp.ones((16,), jnp.float32)
            zero_v = jnp.zeros((16,), jnp.float32)

            def initrow(r, _):
                rows[0, r, pl.ds(16, 16)] = one_v
                for c in range(2, 8):
                    rows[0, r, pl.ds(c * 16, 16)] = zero_v
                return 0

            lax.fori_loop(0, K, initrow, 0)
        plsc.subcore_barrier()

        def load_dln(b, i):
            base = sid * ept + i * K
            pltpu.sync_copy(dst_hbm.at[pl.ds(base, K)], idx_d.at[b])
            for j in range(K // 16):
                d = idx_d[b, pl.ds(j * 16, 16)]
                in_half = (d >= lo) & (d < hi)
                dln[b, pl.ds(j * 16, 16)] = jnp.where(in_half, d - lo, TRASH)

        if ea_pass:
            def chunk(i, _):
                base = sid * ept + i * K
                load_dln(0, i)
                pltpu.sync_copy(ea_hbm.at[pl.ds(base, K)], eav)

                def fill(r, _):
                    rows[0, r, pl.ds(0, EF)] = eav[r, pl.ds(0, EF)]
                    return 0

                lax.fori_loop(0, K, fill, 0)
                pltpu.sync_copy(rows.at[0], acc_h.at[dln.at[0]], add=True)
                return 0

            lax.fori_loop(0, nchunk, chunk, 0)
        else:
            def start(b, i):
                base = sid * ept + i * K
                load_dln(b, i)
                pltpu.sync_copy(src_hbm.at[pl.ds(base, K)], idx_s.at[b])
                pltpu.async_copy(h_hbm.at[idx_s.at[b]], rows.at[b], gsems[b])

            def finish(b):
                pltpu.make_async_copy(h_hbm.at[idx_s.at[b]], rows.at[b],
                                      gsems[b]).wait()
                pltpu.sync_copy(rows.at[b], acc_h.at[dln.at[b]], add=True)

            start(0, 0)

            def pair(g, _):
                i0 = 2 * g

                @pl.when(i0 + 1 < nchunk)
                def _():
                    start(1, i0 + 1)
                finish(0)

                @pl.when(i0 + 2 < nchunk)
                def _():
                    start(0, i0 + 2)

                @pl.when(i0 + 1 < nchunk)
                def _():
                    finish(1)
                return 0

            lax.fori_loop(0, (nchunk + 1) // 2, pair, 0)
        plsc.subcore_barrier()

        @pl.when(sid < NTILE - 1)
        def _():
            pltpu.sync_copy(acc_h.at[pl.ds(sid * OUT_STRIPE, OUT_STRIPE)],
                            out_hbm.at[pl.ds(lo + sid * OUT_STRIPE, OUT_STRIPE)])

        @pl.when(sid == NTILE - 1)
        def _():
            pltpu.sync_copy(
                acc_h.at[pl.ds((NTILE - 1) * OUT_STRIPE, OUT_LAST)],
                out_hbm.at[pl.ds(lo + (NTILE - 1) * OUT_STRIPE, OUT_LAST)])
        plsc.subcore_barrier()


@functools.cache
def _make_segsum(with_edge):
    mesh = plsc.VectorSubcoreMesh(core_axis_name="c", subcore_axis_name="s",
                                  num_cores=NSC, num_subcores=NTILE)
    half = jax.ShapeDtypeStruct((N, COLW), jnp.float32)
    if with_edge:
        out_type = (half, half, half)
    else:
        out_type = (half, half)
    scratch = [
        pltpu.VMEM((2, K), jnp.int32),
        pltpu.VMEM((2, K), jnp.int32),
        pltpu.VMEM((2, K), jnp.int32),
        pltpu.VMEM((2, K, COLW), jnp.float32),
        pltpu.VMEM((K, EF), jnp.float32),
        pltpu.VMEM_SHARED((ACC_ROWS, COLW), jnp.float32),
        pltpu.SemaphoreType.DMA,
        pltpu.SemaphoreType.DMA,
    ]
    return pl.kernel(functools.partial(_segsum_body, with_edge),
                     out_type=out_type, mesh=mesh, scratch_types=scratch,
                     name=f"sage_segsum_{'ea' if with_edge else 'h'}")


def _in_proj_body(x_ref, w_ref, b_ref, o_ref, oa_ref, ob_ref):
    h = jnp.dot(x_ref[...], w_ref[...],
                preferred_element_type=jnp.float32) + b_ref[...]
    o_ref[...] = h
    oa_ref[...] = h[:, :COLW]
    ob_ref[...] = h[:, COLW:]


def _layer_body(h_ref, agga_ref, aggb_ref, eacnt_ref, wnh_ref,
                wne_ref, ws_ref, bb_ref, o_ref, oa_ref, ob_ref):
    eacnt = eacnt_ref[...]
    cnt = eacnt[:, EF:EF + 1] + 1.0
    agg = jnp.concatenate([agga_ref[...], aggb_ref[...]], axis=1)
    pre = (agg + h_ref[...]) / cnt
    q = (jnp.dot(pre, wnh_ref[...], preferred_element_type=jnp.float32)
         + jnp.dot(eacnt[:, :EF] / cnt, wne_ref[...],
                   preferred_element_type=jnp.float32)
         + jnp.dot(h_ref[...], ws_ref[...], preferred_element_type=jnp.float32)
         + bb_ref[...])
    nrm = jnp.sqrt(jnp.sum(q * q, axis=1, keepdims=True))
    out = jnp.maximum(q / jnp.maximum(nrm, 1e-12), 0.0)
    o_ref[...] = out
    oa_ref[...] = out[:, :COLW]
    ob_ref[...] = out[:, COLW:]


def _head_body(hcat_ref, wj_ref, bj_ref, w1_ref, b1_ref, w2_ref, b2_ref,
               wa1_ref, ba1_ref, wa2_ref, ba2_ref, o_ref):
    emb = jnp.dot(hcat_ref[...], wj_ref[...],
                  preferred_element_type=jnp.float32) + bj_ref[...]
    t = jnp.maximum(jnp.dot(emb, w1_ref[...],
                            preferred_element_type=jnp.float32) + b1_ref[...],
                    0.0)
    sc = jnp.sum(t * w2_ref[...].reshape(1, -1), axis=1,
                 keepdims=True) + b2_ref[0, 0]
    ridx = lax.broadcasted_iota(jnp.int32, (16, 1), 0)
    srow = jnp.where(ridx == 0, jnp.float32(-10.0), sc)
    hid = jnp.sum(srow * wa1_ref[...], axis=0, keepdims=True)
    u = jnp.maximum(hid + ba1_ref[...], 0.0)
    o_ref[...] = jnp.dot(u, wa2_ref[...],
                         preferred_element_type=jnp.float32) + ba2_ref[...]


_HALF_SPEC = pl.BlockSpec((ROW_BLK, COLW), lambda i: (i, 0))
_FULL_SPEC = pl.BlockSpec((ROW_BLK, H), lambda i: (i, 0))


def kernel(x, edge_index, edge_attr, W_in, b_in, W_nb0, b_nb0, W_self0, b_self0,
           W_nb1, b_nb1, W_self1, b_self1, W_jump, b_jump,
           W_sc1, b_sc1, W_sc2, b_sc2, W_as1, b_as1, W_as2, b_as2):
    E = edge_index.shape[1]
    ept = ((E + NTILE * K - 1) // (NTILE * K)) * K
    e_pad = ept * NTILE

    src = jnp.concatenate([edge_index[0].astype(jnp.int32),
                           jnp.zeros((e_pad - E,), jnp.int32)])
    dst = jnp.concatenate([edge_index[1].astype(jnp.int32),
                           jnp.full((e_pad - E,), 2**30, jnp.int32)])
    ea = jnp.concatenate([edge_attr,
                          jnp.zeros((e_pad - E, EF), jnp.float32)], axis=0)
    zh = jnp.zeros((ACC_ROWS, COLW), jnp.float32)

    half_sd = jax.ShapeDtypeStruct((N, COLW), jnp.float32)
    full_sd = jax.ShapeDtypeStruct((N, H), jnp.float32)

    h, ha, hb = pl.pallas_call(
        _in_proj_body,
        grid=(N // ROW_BLK,),
        in_specs=[pl.BlockSpec((ROW_BLK, NF), lambda i: (i, 0)),
                  pl.BlockSpec((NF, H), lambda i: (0, 0)),
                  pl.BlockSpec((1, H), lambda i: (0, 0))],
        out_specs=[_FULL_SPEC, _HALF_SPEC, _HALF_SPEC],
        out_shape=[full_sd, half_sd, half_sd],
    )(x, W_in, b_in[None, :])

    def layer(hcur, agga, aggb, eacnt, W_nb, b_nb, W_self, b_self):
        bb = (b_nb + b_self)[None, :]
        return pl.pallas_call(
            _layer_body,
            grid=(N // ROW_BLK,),
            in_specs=[_FULL_SPEC, _HALF_SPEC, _HALF_SPEC, _HALF_SPEC,
                      pl.BlockSpec((H, H), lambda i: (0, 0)),
                      pl.BlockSpec((EF, H), lambda i: (0, 0)),
                      pl.BlockSpec((H, H), lambda i: (0, 0)),
                      pl.BlockSpec((1, H), lambda i: (0, 0))],
            out_specs=[_FULL_SPEC, _HALF_SPEC, _HALF_SPEC],
            out_shape=[full_sd, half_sd, half_sd],
        )(hcur, agga, aggb, eacnt, W_nb[:H], W_nb[H:], W_self, bb)

    agg0a, agg0b, eacnt = _make_segsum(True)(src, dst, ea, ha, hb, zh)
    h1, h1a, h1b = layer(h, agg0a, agg0b, eacnt,
                         W_nb0, b_nb0, W_self0, b_self0)

    agg1a, agg1b = _make_segsum(False)(src, dst, h1a, h1b, zh)
    h2, _, _ = layer(h1, agg1a, agg1b, eacnt,
                     W_nb1, b_nb1, W_self1, b_self1)

    hcat = jnp.concatenate([h1[:16], h2[:16]], axis=1)
    wa1 = jnp.concatenate([W_as1, jnp.zeros((1, H // 2), jnp.float32)], axis=0)
    logits = pl.pallas_call(
        _head_body,
        in_specs=[pl.BlockSpec((16, 2 * H), lambda: (0, 0)),
                  pl.BlockSpec((2 * H, H), lambda: (0, 0)),
                  pl.BlockSpec((1, H), lambda: (0, 0)),
                  pl.BlockSpec((H, H // 2), lambda: (0, 0)),
                  pl.BlockSpec((1, H // 2), lambda: (0, 0)),
                  pl.BlockSpec((H // 2, 1), lambda: (0, 0)),
                  pl.BlockSpec((1, 1), lambda: (0, 0)),
                  pl.BlockSpec((16, H // 2), lambda: (0, 0)),
                  pl.BlockSpec((1, H // 2), lambda: (0, 0)),
                  pl.BlockSpec((H // 2, M), lambda: (0, 0)),
                  pl.BlockSpec((1, M), lambda: (0, 0))],
        out_specs=pl.BlockSpec((1, M), lambda: (0, 0)),
        out_shape=jax.ShapeDtypeStruct((1, M), jnp.float32),
    )(hcat, W_jump, b_jump[None, :], W_sc1, b_sc1[None, :], W_sc2,
      b_sc2[None, :], wa1, b_as1[None, :], W_as2, b_as2[None, :])
    return logits


# speedup vs baseline: 3.9837x; 1.1036x over previous
import functools

import jax
import jax.numpy as jnp
from jax import lax
from jax.experimental import pallas as pl
from jax.experimental.pallas import tpu as pltpu
import jax.experimental.pallas.tpu_sc as plsc

N = 10000
NF = 128
EF = 16
H = 256
M = 15

NSC = 2
NTILE = 16
K = 128
COLW = H // 2
HALF = N // NSC
ACC_ROWS = ((HALF + 8 + 127) // 128) * 128
TRASH = HALF
INIT_STRIPE = ACC_ROWS // NTILE
OUT_STRIPE = (HALF // NTILE) // 8 * 8
OUT_LAST = HALF - (NTILE - 1) * OUT_STRIPE
ROW_BLK = 1000


def _segsum_body(with_edge, *refs):
    if with_edge:
        (src_hbm, dst_hbm, ea_hbm, ha_hbm, hb_hbm, zh_hbm,
         agga_hbm, aggb_hbm, eacnt_hbm,
         idx_s, idx_d, dln, rows, eav, acc_h, sem0, sem1) = refs
    else:
        (src_hbm, dst_hbm, ha_hbm, hb_hbm, zh_hbm,
         agga_hbm, aggb_hbm,
         idx_s, idx_d, dln, rows, eav, acc_h, sem0, sem1) = refs
    gsems = (sem0, sem1)

    cid = lax.axis_index("c")
    sid = lax.axis_index("s")
    lo = cid * HALF
    hi = lo + HALF
    r0 = sid * INIT_STRIPE

    ept = src_hbm.shape[0] // NTILE
    nchunk = ept // K

    passes = [(ha_hbm, agga_hbm, False), (hb_hbm, aggb_hbm, False)]
    if with_edge:
        passes.append((ha_hbm, eacnt_hbm, True))

    for h_hbm, out_hbm, ea_pass in passes:
        pltpu.sync_copy(zh_hbm.at[pl.ds(r0, INIT_STRIPE)],
                        acc_h.at[pl.ds(r0, INIT_STRIPE)])
        if ea_pass:
            one_v = jnp.ones((16,), jnp.float32)
            zero_v = jnp.zeros((16,), jnp.float32)

            def initrow(r, _):
                rows[0, r, pl.ds(16, 16)] = one_v
                for c in range(2, 8):
                    rows[0, r, pl.ds(c * 16, 16)] = zero_v
                return 0

            lax.fori_loop(0, K, initrow, 0)
        plsc.subcore_barrier()

        def dln_compute(b):
            for j in range(K // 16):
                d = idx_d[b, pl.ds(j * 16, 16)]
                in_half = (d >= lo) & (d < hi)
                dln[b, pl.ds(j * 16, 16)] = jnp.where(in_half, d - lo, TRASH)

        if ea_pass:
            def start_e(b, i):
                base = sid * ept + i * K
                pltpu.async_copy(dst_hbm.at[pl.ds(base, K)], idx_d.at[b],
                                 gsems[b])
                pltpu.async_copy(ea_hbm.at[pl.ds(base, K)], eav.at[b],
                                 gsems[b])

            def finish_e(b, i):
                base = sid * ept + i * K
                pltpu.make_async_copy(dst_hbm.at[pl.ds(base, K)],
                                      idx_d.at[b], gsems[b]).wait()
                pltpu.make_async_copy(ea_hbm.at[pl.ds(base, K)],
                                      eav.at[b], gsems[b]).wait()
                dln_compute(b)

                def fill(r, _):
                    rows[0, r, pl.ds(0, EF)] = eav[b, r, pl.ds(0, EF)]
                    return 0

                lax.fori_loop(0, K, fill, 0)
                pltpu.sync_copy(rows.at[0], acc_h.at[dln.at[b]], add=True)

            start_e(0, 0)

            def pair_e(g, _):
                i0 = 2 * g

                @pl.when(i0 + 1 < nchunk)
                def _():
                    start_e(1, i0 + 1)
                finish_e(0, i0)

                @pl.when(i0 + 2 < nchunk)
                def _():
                    start_e(0, i0 + 2)

                @pl.when(i0 + 1 < nchunk)
                def _():
                    finish_e(1, i0 + 1)
                return 0

            lax.fori_loop(0, (nchunk + 1) // 2, pair_e, 0)
        else:
            def start(b, i):
                base = sid * ept + i * K
                pltpu.sync_copy(dst_hbm.at[pl.ds(base, K)], idx_d.at[b])
                dln_compute(b)
                pltpu.sync_copy(src_hbm.at[pl.ds(base, K)], idx_s.at[b])
                pltpu.async_copy(h_hbm.at[idx_s.at[b]], rows.at[b], gsems[b])

            def finish(b):
                pltpu.make_async_copy(h_hbm.at[idx_s.at[b]], rows.at[b],
                                      gsems[b]).wait()
                pltpu.sync_copy(rows.at[b], acc_h.at[dln.at[b]], add=True)

            start(0, 0)

            def pair(g, _):
                i0 = 2 * g

                @pl.when(i0 + 1 < nchunk)
                def _():
                    start(1, i0 + 1)
                finish(0)

                @pl.when(i0 + 2 < nchunk)
                def _():
                    start(0, i0 + 2)

                @pl.when(i0 + 1 < nchunk)
                def _():
                    finish(1)
                return 0

            lax.fori_loop(0, (nchunk + 1) // 2, pair, 0)
        plsc.subcore_barrier()

        @pl.when(sid < NTILE - 1)
        def _():
            pltpu.sync_copy(acc_h.at[pl.ds(sid * OUT_STRIPE, OUT_STRIPE)],
                            out_hbm.at[pl.ds(lo + sid * OUT_STRIPE, OUT_STRIPE)])

        @pl.when(sid == NTILE - 1)
        def _():
            pltpu.sync_copy(
                acc_h.at[pl.ds((NTILE - 1) * OUT_STRIPE, OUT_LAST)],
                out_hbm.at[pl.ds(lo + (NTILE - 1) * OUT_STRIPE, OUT_LAST)])
        plsc.subcore_barrier()


@functools.cache
def _make_segsum(with_edge, ept):
    mesh = plsc.VectorSubcoreMesh(core_axis_name="c", subcore_axis_name="s",
                                  num_cores=NSC, num_subcores=NTILE)
    half = jax.ShapeDtypeStruct((N, COLW), jnp.float32)
    if with_edge:
        out_type = (half, half, half)
    else:
        out_type = (half, half)
    scratch = [
        pltpu.VMEM((2, K), jnp.int32),
        pltpu.VMEM((2, K), jnp.int32),
        pltpu.VMEM((2, K), jnp.int32),
        pltpu.VMEM((2, K, COLW), jnp.float32),
        pltpu.VMEM((2, K, EF), jnp.float32),
        pltpu.VMEM_SHARED((ACC_ROWS, COLW), jnp.float32),
        pltpu.SemaphoreType.DMA,
        pltpu.SemaphoreType.DMA,
    ]
    return pl.kernel(functools.partial(_segsum_body, with_edge),
                     out_type=out_type, mesh=mesh, scratch_types=scratch,
                     name=f"sage_segsum_{'ea' if with_edge else 'h'}")


def _in_proj_body(x_ref, w_ref, b_ref, o_ref, oa_ref, ob_ref):
    h = jnp.dot(x_ref[...], w_ref[...],
                preferred_element_type=jnp.float32) + b_ref[...]
    o_ref[...] = h
    oa_ref[...] = h[:, :COLW]
    ob_ref[...] = h[:, COLW:]


def _layer_body(h_ref, agga_ref, aggb_ref, eacnt_ref, wnh_ref,
                wne_ref, ws_ref, bb_ref, o_ref, oa_ref, ob_ref):
    eacnt = eacnt_ref[...]
    cnt = eacnt[:, EF:EF + 1] + 1.0
    agg = jnp.concatenate([agga_ref[...], aggb_ref[...]], axis=1)
    pre = (agg + h_ref[...]) / cnt
    q = (jnp.dot(pre, wnh_ref[...], preferred_element_type=jnp.float32)
         + jnp.dot(eacnt[:, :EF] / cnt, wne_ref[...],
                   preferred_element_type=jnp.float32)
         + jnp.dot(h_ref[...], ws_ref[...], preferred_element_type=jnp.float32)
         + bb_ref[...])
    nrm = jnp.sqrt(jnp.sum(q * q, axis=1, keepdims=True))
    out = jnp.maximum(q / jnp.maximum(nrm, 1e-12), 0.0)
    o_ref[...] = out
    oa_ref[...] = out[:, :COLW]
    ob_ref[...] = out[:, COLW:]


def _head_body(hcat_ref, wj_ref, bj_ref, w1_ref, b1_ref, w2_ref, b2_ref,
               wa1_ref, ba1_ref, wa2_ref, ba2_ref, o_ref):
    emb = jnp.dot(hcat_ref[...], wj_ref[...],
                  preferred_element_type=jnp.float32) + bj_ref[...]
    t = jnp.maximum(jnp.dot(emb, w1_ref[...],
                            preferred_element_type=jnp.float32) + b1_ref[...],
                    0.0)
    sc = jnp.sum(t * w2_ref[...].reshape(1, -1), axis=1,
                 keepdims=True) + b2_ref[0, 0]
    ridx = lax.broadcasted_iota(jnp.int32, (16, 1), 0)
    srow = jnp.where(ridx == 0, jnp.float32(-10.0), sc)
    hid = jnp.sum(srow * wa1_ref[...], axis=0, keepdims=True)
    u = jnp.maximum(hid + ba1_ref[...], 0.0)
    o_ref[...] = jnp.dot(u, wa2_ref[...],
                         preferred_element_type=jnp.float32) + ba2_ref[...]


_HALF_SPEC = pl.BlockSpec((ROW_BLK, COLW), lambda i: (i, 0))
_FULL_SPEC = pl.BlockSpec((ROW_BLK, H), lambda i: (i, 0))


def kernel(x, edge_index, edge_attr, W_in, b_in, W_nb0, b_nb0, W_self0, b_self0,
           W_nb1, b_nb1, W_self1, b_self1, W_jump, b_jump,
           W_sc1, b_sc1, W_sc2, b_sc2, W_as1, b_as1, W_as2, b_as2):
    E = edge_index.shape[1]
    ept = ((E + NTILE * K - 1) // (NTILE * K)) * K
    e_pad = ept * NTILE

    src = jnp.concatenate([edge_index[0].astype(jnp.int32),
                           jnp.zeros((e_pad - E,), jnp.int32)])
    dst = jnp.concatenate([edge_index[1].astype(jnp.int32),
                           jnp.full((e_pad - E,), 2**30, jnp.int32)])
    ea = jnp.concatenate([edge_attr,
                          jnp.zeros((e_pad - E, EF), jnp.float32)], axis=0)
    zh = jnp.zeros((ACC_ROWS, COLW), jnp.float32)

    half_sd = jax.ShapeDtypeStruct((N, COLW), jnp.float32)
    full_sd = jax.ShapeDtypeStruct((N, H), jnp.float32)

    h, ha, hb = pl.pallas_call(
        _in_proj_body,
        grid=(N // ROW_BLK,),
        in_specs=[pl.BlockSpec((ROW_BLK, NF), lambda i: (i, 0)),
                  pl.BlockSpec((NF, H), lambda i: (0, 0)),
                  pl.BlockSpec((1, H), lambda i: (0, 0))],
        out_specs=[_FULL_SPEC, _HALF_SPEC, _HALF_SPEC],
        out_shape=[full_sd, half_sd, half_sd],
    )(x, W_in, b_in[None, :])

    def layer(hcur, agga, aggb, eacnt, W_nb, b_nb, W_self, b_self):
        bb = (b_nb + b_self)[None, :]
        return pl.pallas_call(
            _layer_body,
            grid=(N // ROW_BLK,),
            in_specs=[_FULL_SPEC, _HALF_SPEC, _HALF_SPEC, _HALF_SPEC,
                      pl.BlockSpec((H, H), lambda i: (0, 0)),
                      pl.BlockSpec((EF, H), lambda i: (0, 0)),
                      pl.BlockSpec((H, H), lambda i: (0, 0)),
                      pl.BlockSpec((1, H), lambda i: (0, 0))],
            out_specs=[_FULL_SPEC, _HALF_SPEC, _HALF_SPEC],
            out_shape=[full_sd, half_sd, half_sd],
        )(hcur, agga, aggb, eacnt, W_nb[:H], W_nb[H:], W_self, bb)

    agg0a, agg0b, eacnt = _make_segsum(True, ept)(src, dst, ea, ha, hb, zh)
    h1, h1a, h1b = layer(h, agg0a, agg0b, eacnt,
                         W_nb0, b_nb0, W_self0, b_self0)

    agg1a, agg1b = _make_segsum(False, ept)(src, dst, h1a, h1b, zh)
    h2, _, _ = layer(h1, agg1a, agg1b, eacnt,
                     W_nb1, b_nb1, W_self1, b_self1)

    hcat = jnp.concatenate([h1[:16], h2[:16]], axis=1)
    wa1 = jnp.concatenate([W_as1, jnp.zeros((1, H // 2), jnp.float32)], axis=0)
    logits = pl.pallas_call(
        _head_body,
        in_specs=[pl.BlockSpec((16, 2 * H), lambda: (0, 0)),
                  pl.BlockSpec((2 * H, H), lambda: (0, 0)),
                  pl.BlockSpec((1, H), lambda: (0, 0)),
                  pl.BlockSpec((H, H // 2), lambda: (0, 0)),
                  pl.BlockSpec((1, H // 2), lambda: (0, 0)),
                  pl.BlockSpec((H // 2, 1), lambda: (0, 0)),
                  pl.BlockSpec((1, 1), lambda: (0, 0)),
                  pl.BlockSpec((16, H // 2), lambda: (0, 0)),
                  pl.BlockSpec((1, H // 2), lambda: (0, 0)),
                  pl.BlockSpec((H // 2, M), lambda: (0, 0)),
                  pl.BlockSpec((1, M), lambda: (0, 0))],
        out_specs=pl.BlockSpec((1, M), lambda: (0, 0)),
        out_shape=jax.ShapeDtypeStruct((1, M), jnp.float32),
    )(hcat, W_jump, b_jump[None, :], W_sc1, b_sc1[None, :], W_sc2,
      b_sc2[None, :], wa1, b_as1[None, :], W_as2, b_as2[None, :])
    return logits


# speedup vs baseline: 4.2649x; 1.0706x over previous
import functools

import jax
import jax.numpy as jnp
from jax import lax
from jax.experimental import pallas as pl
from jax.experimental.pallas import tpu as pltpu
import jax.experimental.pallas.tpu_sc as plsc

N = 10000
NF = 128
EF = 16
H = 256
M = 15

NSC = 2
NTILE = 16
K = 128
COLW = H // 2
HALF = N // NSC
ACC_ROWS = ((HALF + 8 + 127) // 128) * 128
TRASH = HALF
INIT_STRIPE = ACC_ROWS // NTILE
OUT_STRIPE = (HALF // NTILE) // 8 * 8
OUT_LAST = HALF - (NTILE - 1) * OUT_STRIPE
ROW_BLK = 1000


def _segsum_body(with_edge, *refs):
    if with_edge:
        (src_hbm, dst_hbm, ea_hbm, ha_hbm, hb_hbm, zh_hbm,
         agga_hbm, aggb_hbm, eacnt_hbm,
         idx_s, idx_d, dln, rows, eav, acc_h, sem0, sem1, sem2) = refs
    else:
        (src_hbm, dst_hbm, ha_hbm, hb_hbm, zh_hbm,
         agga_hbm, aggb_hbm,
         idx_s, idx_d, dln, rows, eav, acc_h, sem0, sem1, sem2) = refs
    gsems = (sem0, sem1, sem2)

    cid = lax.axis_index("c")
    sid = lax.axis_index("s")
    lo = cid * HALF
    hi = lo + HALF
    r0 = sid * INIT_STRIPE

    ept = src_hbm.shape[0] // NTILE
    nchunk = ept // K

    passes = [(ha_hbm, agga_hbm, False), (hb_hbm, aggb_hbm, False)]
    if with_edge:
        passes.append((ha_hbm, eacnt_hbm, True))

    for h_hbm, out_hbm, ea_pass in passes:
        pltpu.sync_copy(zh_hbm.at[pl.ds(r0, INIT_STRIPE)],
                        acc_h.at[pl.ds(r0, INIT_STRIPE)])
        if ea_pass:
            one_v = jnp.ones((16,), jnp.float32)
            zero_v = jnp.zeros((16,), jnp.float32)

            def initrow(r, _):
                rows[0, r, pl.ds(16, 16)] = one_v
                for c in range(2, 8):
                    rows[0, r, pl.ds(c * 16, 16)] = zero_v
                return 0

            lax.fori_loop(0, K, initrow, 0)
        plsc.subcore_barrier()

        def dln_compute(b):
            for j in range(K // 16):
                d = idx_d[b, pl.ds(j * 16, 16)]
                in_half = (d >= lo) & (d < hi)
                dln[b, pl.ds(j * 16, 16)] = jnp.where(in_half, d - lo, TRASH)

        if ea_pass:
            def start_e(b, i):
                base = sid * ept + i * K
                pltpu.async_copy(dst_hbm.at[pl.ds(base, K)], idx_d.at[b],
                                 gsems[b])
                pltpu.async_copy(ea_hbm.at[pl.ds(base, K)], eav.at[b],
                                 gsems[b])

            def finish_e(b, i):
                base = sid * ept + i * K
                pltpu.make_async_copy(dst_hbm.at[pl.ds(base, K)],
                                      idx_d.at[b], gsems[b]).wait()
                pltpu.make_async_copy(ea_hbm.at[pl.ds(base, K)],
                                      eav.at[b], gsems[b]).wait()
                dln_compute(b)

                def fill(r, _):
                    rows[0, r, pl.ds(0, EF)] = eav[b, r, pl.ds(0, EF)]
                    return 0

                lax.fori_loop(0, K, fill, 0)
                pltpu.sync_copy(rows.at[0], acc_h.at[dln.at[b]], add=True)

            start_e(0, 0)

            def pair_e(g, _):
                i0 = 2 * g

                @pl.when(i0 + 1 < nchunk)
                def _():
                    start_e(1, i0 + 1)
                finish_e(0, i0)

                @pl.when(i0 + 2 < nchunk)
                def _():
                    start_e(0, i0 + 2)

                @pl.when(i0 + 1 < nchunk)
                def _():
                    finish_e(1, i0 + 1)
                return 0

            lax.fori_loop(0, (nchunk + 1) // 2, pair_e, 0)
        else:
            def s1(b, i):
                base = sid * ept + i * K
                pltpu.async_copy(dst_hbm.at[pl.ds(base, K)], idx_d.at[b],
                                 gsems[b])
                pltpu.async_copy(src_hbm.at[pl.ds(base, K)], idx_s.at[b],
                                 gsems[b])

            def s2(b, i):
                base = sid * ept + i * K
                pltpu.make_async_copy(dst_hbm.at[pl.ds(base, K)],
                                      idx_d.at[b], gsems[b]).wait()
                pltpu.make_async_copy(src_hbm.at[pl.ds(base, K)],
                                      idx_s.at[b], gsems[b]).wait()
                dln_compute(b)
                pltpu.async_copy(h_hbm.at[idx_s.at[b]], rows.at[b], gsems[b])

            def s3(b):
                pltpu.make_async_copy(h_hbm.at[idx_s.at[b]], rows.at[b],
                                      gsems[b]).wait()
                pltpu.sync_copy(rows.at[b], acc_h.at[dln.at[b]], add=True)

            s1(0, 0)

            @pl.when(1 < nchunk)
            def _():
                s1(1, 1)
            s2(0, 0)

            def tri(g, _):
                i0 = 3 * g
                for u in range(3):
                    i = i0 + u

                    @pl.when(i + 2 < nchunk)
                    def _():
                        s1((u + 2) % 3, i + 2)

                    @pl.when(i + 1 < nchunk)
                    def _():
                        s2((u + 1) % 3, i + 1)

                    @pl.when(i < nchunk)
                    def _():
                        s3(u)
                return 0

            lax.fori_loop(0, (nchunk + 2) // 3, tri, 0)
        plsc.subcore_barrier()

        @pl.when(sid < NTILE - 1)
        def _():
            pltpu.sync_copy(acc_h.at[pl.ds(sid * OUT_STRIPE, OUT_STRIPE)],
                            out_hbm.at[pl.ds(lo + sid * OUT_STRIPE, OUT_STRIPE)])

        @pl.when(sid == NTILE - 1)
        def _():
            pltpu.sync_copy(
                acc_h.at[pl.ds((NTILE - 1) * OUT_STRIPE, OUT_LAST)],
                out_hbm.at[pl.ds(lo + (NTILE - 1) * OUT_STRIPE, OUT_LAST)])
        plsc.subcore_barrier()


@functools.cache
def _make_segsum(with_edge, ept):
    mesh = plsc.VectorSubcoreMesh(core_axis_name="c", subcore_axis_name="s",
                                  num_cores=NSC, num_subcores=NTILE)
    half = jax.ShapeDtypeStruct((N, COLW), jnp.float32)
    if with_edge:
        out_type = (half, half, half)
    else:
        out_type = (half, half)
    scratch = [
        pltpu.VMEM((3, K), jnp.int32),
        pltpu.VMEM((3, K), jnp.int32),
        pltpu.VMEM((3, K), jnp.int32),
        pltpu.VMEM((3, K, COLW), jnp.float32),
        pltpu.VMEM((2, K, EF), jnp.float32),
        pltpu.VMEM_SHARED((ACC_ROWS, COLW), jnp.float32),
        pltpu.SemaphoreType.DMA,
        pltpu.SemaphoreType.DMA,
        pltpu.SemaphoreType.DMA,
    ]
    return pl.kernel(functools.partial(_segsum_body, with_edge),
                     out_type=out_type, mesh=mesh, scratch_types=scratch,
                     name=f"sage_segsum_{'ea' if with_edge else 'h'}")


def _in_proj_body(x_ref, w_ref, b_ref, o_ref, oa_ref, ob_ref):
    h = jnp.dot(x_ref[...], w_ref[...],
                preferred_element_type=jnp.float32) + b_ref[...]
    o_ref[...] = h
    oa_ref[...] = h[:, :COLW]
    ob_ref[...] = h[:, COLW:]


def _layer_body(h_ref, agga_ref, aggb_ref, eacnt_ref, wnh_ref,
                wne_ref, ws_ref, bb_ref, o_ref, oa_ref, ob_ref):
    eacnt = eacnt_ref[...]
    cnt = eacnt[:, EF:EF + 1] + 1.0
    agg = jnp.concatenate([agga_ref[...], aggb_ref[...]], axis=1)
    pre = (agg + h_ref[...]) / cnt
    q = (jnp.dot(pre, wnh_ref[...], preferred_element_type=jnp.float32)
         + jnp.dot(eacnt[:, :EF] / cnt, wne_ref[...],
                   preferred_element_type=jnp.float32)
         + jnp.dot(h_ref[...], ws_ref[...], preferred_element_type=jnp.float32)
         + bb_ref[...])
    nrm = jnp.sqrt(jnp.sum(q * q, axis=1, keepdims=True))
    out = jnp.maximum(q / jnp.maximum(nrm, 1e-12), 0.0)
    o_ref[...] = out
    oa_ref[...] = out[:, :COLW]
    ob_ref[...] = out[:, COLW:]


def _head_body(hcat_ref, wj_ref, bj_ref, w1_ref, b1_ref, w2_ref, b2_ref,
               wa1_ref, ba1_ref, wa2_ref, ba2_ref, o_ref):
    emb = jnp.dot(hcat_ref[...], wj_ref[...],
                  preferred_element_type=jnp.float32) + bj_ref[...]
    t = jnp.maximum(jnp.dot(emb, w1_ref[...],
                            preferred_element_type=jnp.float32) + b1_ref[...],
                    0.0)
    sc = jnp.sum(t * w2_ref[...].reshape(1, -1), axis=1,
                 keepdims=True) + b2_ref[0, 0]
    ridx = lax.broadcasted_iota(jnp.int32, (16, 1), 0)
    srow = jnp.where(ridx == 0, jnp.float32(-10.0), sc)
    hid = jnp.sum(srow * wa1_ref[...], axis=0, keepdims=True)
    u = jnp.maximum(hid + ba1_ref[...], 0.0)
    o_ref[...] = jnp.dot(u, wa2_ref[...],
                         preferred_element_type=jnp.float32) + ba2_ref[...]


_HALF_SPEC = pl.BlockSpec((ROW_BLK, COLW), lambda i: (i, 0))
_FULL_SPEC = pl.BlockSpec((ROW_BLK, H), lambda i: (i, 0))


def kernel(x, edge_index, edge_attr, W_in, b_in, W_nb0, b_nb0, W_self0, b_self0,
           W_nb1, b_nb1, W_self1, b_self1, W_jump, b_jump,
           W_sc1, b_sc1, W_sc2, b_sc2, W_as1, b_as1, W_as2, b_as2):
    E = edge_index.shape[1]
    ept = ((E + NTILE * K - 1) // (NTILE * K)) * K
    e_pad = ept * NTILE

    src = jnp.concatenate([edge_index[0].astype(jnp.int32),
                           jnp.zeros((e_pad - E,), jnp.int32)])
    dst = jnp.concatenate([edge_index[1].astype(jnp.int32),
                           jnp.full((e_pad - E,), 2**30, jnp.int32)])
    ea = jnp.concatenate([edge_attr,
                          jnp.zeros((e_pad - E, EF), jnp.float32)], axis=0)
    zh = jnp.zeros((ACC_ROWS, COLW), jnp.float32)

    half_sd = jax.ShapeDtypeStruct((N, COLW), jnp.float32)
    full_sd = jax.ShapeDtypeStruct((N, H), jnp.float32)

    h, ha, hb = pl.pallas_call(
        _in_proj_body,
        grid=(N // ROW_BLK,),
        in_specs=[pl.BlockSpec((ROW_BLK, NF), lambda i: (i, 0)),
                  pl.BlockSpec((NF, H), lambda i: (0, 0)),
                  pl.BlockSpec((1, H), lambda i: (0, 0))],
        out_specs=[_FULL_SPEC, _HALF_SPEC, _HALF_SPEC],
        out_shape=[full_sd, half_sd, half_sd],
    )(x, W_in, b_in[None, :])

    def layer(hcur, agga, aggb, eacnt, W_nb, b_nb, W_self, b_self):
        bb = (b_nb + b_self)[None, :]
        return pl.pallas_call(
            _layer_body,
            grid=(N // ROW_BLK,),
            in_specs=[_FULL_SPEC, _HALF_SPEC, _HALF_SPEC, _HALF_SPEC,
                      pl.BlockSpec((H, H), lambda i: (0, 0)),
                      pl.BlockSpec((EF, H), lambda i: (0, 0)),
                      pl.BlockSpec((H, H), lambda i: (0, 0)),
                      pl.BlockSpec((1, H), lambda i: (0, 0))],
            out_specs=[_FULL_SPEC, _HALF_SPEC, _HALF_SPEC],
            out_shape=[full_sd, half_sd, half_sd],
        )(hcur, agga, aggb, eacnt, W_nb[:H], W_nb[H:], W_self, bb)

    agg0a, agg0b, eacnt = _make_segsum(True, ept)(src, dst, ea, ha, hb, zh)
    h1, h1a, h1b = layer(h, agg0a, agg0b, eacnt,
                         W_nb0, b_nb0, W_self0, b_self0)

    agg1a, agg1b = _make_segsum(False, ept)(src, dst, h1a, h1b, zh)
    h2, _, _ = layer(h1, agg1a, agg1b, eacnt,
                     W_nb1, b_nb1, W_self1, b_self1)

    hcat = jnp.concatenate([h1[:16], h2[:16]], axis=1)
    wa1 = jnp.concatenate([W_as1, jnp.zeros((1, H // 2), jnp.float32)], axis=0)
    logits = pl.pallas_call(
        _head_body,
        in_specs=[pl.BlockSpec((16, 2 * H), lambda: (0, 0)),
                  pl.BlockSpec((2 * H, H), lambda: (0, 0)),
                  pl.BlockSpec((1, H), lambda: (0, 0)),
                  pl.BlockSpec((H, H // 2), lambda: (0, 0)),
                  pl.BlockSpec((1, H // 2), lambda: (0, 0)),
                  pl.BlockSpec((H // 2, 1), lambda: (0, 0)),
                  pl.BlockSpec((1, 1), lambda: (0, 0)),
                  pl.BlockSpec((16, H // 2), lambda: (0, 0)),
                  pl.BlockSpec((1, H // 2), lambda: (0, 0)),
                  pl.BlockSpec((H // 2, M), lambda: (0, 0)),
                  pl.BlockSpec((1, M), lambda: (0, 0))],
        out_specs=pl.BlockSpec((1, M), lambda: (0, 0)),
        out_shape=jax.ShapeDtypeStruct((1, M), jnp.float32),
    )(hcat, W_jump, b_jump[None, :], W_sc1, b_sc1[None, :], W_sc2,
      b_sc2[None, :], wa1, b_as1[None, :], W_as2, b_as2[None, :])
    return logits
